# baseline (speedup 1.0000x reference)
import jax
import jax.numpy as jnp
from jax import lax
from jax.experimental import pallas as pl
from jax.experimental.pallas import tpu as pltpu
from jax.experimental.pallas import tpu_sc as plsc

_BATCH = 100
_DIM = 1024
_MARGIN = 0.1
_NS = 16
_RPW = 8
_HALF = _RPW // 2
_PAD = _NS * _RPW
_LANES = 16
_CHUNKS = _DIM // _LANES

_mesh = plsc.VectorSubcoreMesh(
    core_axis_name="c", subcore_axis_name="s", num_cores=1, num_subcores=_NS
)

_SCRATCH = [
    pltpu.VMEM((2 * _RPW,), jnp.int32),
    pltpu.VMEM((_RPW, _DIM), jnp.float32),
    pltpu.VMEM((_RPW, _DIM), jnp.float32),
    pltpu.VMEM((_RPW, _DIM), jnp.float32),
    pltpu.VMEM((_RPW, _LANES), jnp.float32),
    pltpu.SemaphoreType.DMA,
    pltpu.SemaphoreType.DMA,
    pltpu.SemaphoreType.DMA,
]


def _half_accum(outs, rows, drows, row0):
    zero = jnp.zeros((_LANES,), jnp.float32)

    def body(j, accs):
        col = j * _LANES
        new = []
        for r in range(_HALF):
            o = outs[row0 + r, pl.ds(col, _LANES)]
            t = rows[r, pl.ds(col, _LANES)]
            n = rows[r + _HALF, pl.ds(col, _LANES)]
            new.append(accs[r] + o * (n - t))
        return tuple(new)

    accs = lax.fori_loop(0, _CHUNKS, body, (zero,) * _HALF)
    for r in range(_HALF):
        drows[row0 + r] = accs[r]


def _loss_body(outputs_hbm, idx_hbm, emb_hbm, out_hbm,
               idx_v, outs, rows_a, rows_b, drows, sem_a, sem_b, sem_o):
    sid = lax.axis_index("s")
    base = sid * _RPW
    cp_o = pltpu.async_copy(outputs_hbm.at[pl.ds(base, _RPW)], outs, sem_o)
    pltpu.sync_copy(idx_hbm.at[sid], idx_v)
    cp_a = pltpu.async_copy(emb_hbm.at[idx_v.at[pl.ds(0, _RPW)]], rows_a, sem_a)
    cp_b = pltpu.async_copy(emb_hbm.at[idx_v.at[pl.ds(_RPW, _RPW)]], rows_b, sem_b)
    cp_o.wait()
    cp_a.wait()
    _half_accum(outs, rows_a, drows, 0)
    cp_w = pltpu.async_copy(drows.at[pl.ds(0, _HALF)],
                            out_hbm.at[sid, pl.ds(0, _HALF)], sem_a)
    cp_b.wait()
    _half_accum(outs, rows_b, drows, _HALF)
    cp_w.wait()
    pltpu.sync_copy(drows.at[pl.ds(_HALF, _HALF)],
                    out_hbm.at[sid, pl.ds(_HALF, _HALF)])


_loss_kernel = pl.kernel(
    _loss_body,
    out_type=jax.ShapeDtypeStruct((_NS, _RPW, _LANES), jnp.float32),
    mesh=_mesh,
    scratch_types=_SCRATCH,
)


def _sum_body(parts_ref, out_ref):
    d = jnp.sum(parts_ref[...], axis=1)
    loss = jnp.maximum(jnp.float32(_MARGIN) + d, 0.0)
    row = lax.broadcasted_iota(jnp.int32, (_PAD,), 0)
    loss = jnp.where(row < _BATCH, loss, 0.0)
    out_ref[0, 0] = jnp.sum(loss)


_sum_kernel = pl.pallas_call(
    _sum_body,
    out_shape=jax.ShapeDtypeStruct((1, 1), jnp.float32),
    in_specs=[pl.BlockSpec(memory_space=pltpu.VMEM)],
    out_specs=pl.BlockSpec(memory_space=pltpu.SMEM),
)


def kernel(outputs, labels, labels_random, embeddings):
    pad = _PAD - _BATCH
    outputs_p = jnp.pad(outputs, ((0, pad), (0, 0)))
    labels_p = jnp.pad(labels, (0, pad)).reshape(_NS, 2, _HALF)
    rand_p = jnp.pad(labels_random, (0, pad)).reshape(_NS, 2, _HALF)
    idx = jnp.concatenate([labels_p, rand_p], axis=2).reshape(_NS, 2 * _RPW)
    parts = _loss_kernel(outputs_p, idx, embeddings)
    return _sum_kernel(parts.reshape(_PAD, _LANES))[0, 0]


# speedup vs baseline: 1.0354x; 1.0354x over previous
import jax
import jax.numpy as jnp
from jax import lax
from jax.experimental import pallas as pl
from jax.experimental.pallas import tpu as pltpu
from jax.experimental.pallas import tpu_sc as plsc

_BATCH = 100
_DIM = 1024
_MARGIN = 0.1
_NS = 16
_RPW = 8
_PAD = _NS * _RPW
_LANES = 16
_CHUNKS = _DIM // _LANES

_mesh = plsc.VectorSubcoreMesh(
    core_axis_name="c", subcore_axis_name="s", num_cores=1, num_subcores=_NS
)

_SCRATCH = [
    pltpu.VMEM((2 * _RPW,), jnp.int32),
    pltpu.VMEM((_RPW, _DIM), jnp.float32),
    pltpu.VMEM((2 * _RPW, _DIM), jnp.float32),
    pltpu.VMEM((_RPW, _LANES), jnp.float32),
    pltpu.SemaphoreType.DMA,
    pltpu.SemaphoreType.DMA,
]


def _loss_body(outputs_hbm, idx_hbm, emb_hbm, out_hbm,
               idx_v, outs, rows, drows, sem_g, sem_o):
    sid = lax.axis_index("s")
    base = sid * _RPW
    cp_o = pltpu.async_copy(outputs_hbm.at[pl.ds(base, _RPW)], outs, sem_o)
    pltpu.sync_copy(idx_hbm.at[sid], idx_v)
    cp_g = pltpu.async_copy(emb_hbm.at[idx_v], rows, sem_g)
    cp_o.wait()
    cp_g.wait()

    zero = jnp.zeros((_LANES,), jnp.float32)

    @plsc.parallel_loop(0, _CHUNKS, unroll=2, carry=(zero,) * _RPW)
    def accs(j, accs_in):
        col = j * _LANES
        new = []
        for r in range(_RPW):
            o = outs[r, pl.ds(col, _LANES)]
            t = rows[r, pl.ds(col, _LANES)]
            n = rows[r + _RPW, pl.ds(col, _LANES)]
            new.append(accs_in[r] + o * (n - t))
        return tuple(new)
    for r in range(_RPW):
        drows[r] = accs[r]
    pltpu.sync_copy(drows, out_hbm.at[sid])


_loss_kernel = pl.kernel(
    _loss_body,
    out_type=jax.ShapeDtypeStruct((_NS, _RPW, _LANES), jnp.float32),
    mesh=_mesh,
    scratch_types=_SCRATCH,
)


def _sum_body(parts_ref, out_ref):
    d = jnp.sum(parts_ref[...], axis=1)
    loss = jnp.maximum(jnp.float32(_MARGIN) + d, 0.0)
    row = lax.broadcasted_iota(jnp.int32, (_PAD,), 0)
    loss = jnp.where(row < _BATCH, loss, 0.0)
    out_ref[0, 0] = jnp.sum(loss)


_sum_kernel = pl.pallas_call(
    _sum_body,
    out_shape=jax.ShapeDtypeStruct((1, 1), jnp.float32),
    in_specs=[pl.BlockSpec(memory_space=pltpu.VMEM)],
    out_specs=pl.BlockSpec(memory_space=pltpu.SMEM),
)


def kernel(outputs, labels, labels_random, embeddings):
    pad = _PAD - _BATCH
    outputs_p = jnp.pad(outputs, ((0, pad), (0, 0)))
    labels_p = jnp.pad(labels, (0, pad)).reshape(_NS, _RPW)
    rand_p = jnp.pad(labels_random, (0, pad)).reshape(_NS, _RPW)
    idx = jnp.concatenate([labels_p, rand_p], axis=1)
    parts = _loss_kernel(outputs_p, idx, embeddings)
    return _sum_kernel(parts.reshape(_PAD, _LANES))[0, 0]
